# baseline (speedup 1.0000x reference)
import functools

import jax
import jax.numpy as jnp
from jax import lax
from jax.experimental import pallas as pl
from jax.experimental.pallas import tpu as pltpu
from jax.experimental.pallas import tpu_sc as plsc

_N = 50000
_E = 800000
_H = 64
_NB = 400
_EB = 640
_HALF = _N // 2
_HPAD = 25088
_STRIPE = _HPAD // 16
_LAST = _HALF - 15 * _STRIPE
_TRASH0 = 25008
_CH = 64
_EPAD = 819200
_NVB = _E // 640
_EPW = _EPAD // 16
_NCH = _EPW // _CH
_ZR = 49
_EPS = 1e-5
_F32 = jnp.float32
_BF = jnp.bfloat16
_PI = ([2 * i for i in range(16)] + [2 * i + 1 for i in range(16)]
       + [32 + 2 * i for i in range(16)] + [33 + 2 * i for i in range(16)])


def _ln(h, g, b):
    mu = jnp.mean(h, axis=-1, keepdims=True)
    d = h - mu
    var = jnp.mean(d * d, axis=-1, keepdims=True)
    return d * lax.rsqrt(var + _EPS) * g + b


def _mlp3(x, W0, b0, g0, p0, W1, b1, g1, p1, W2, b2):
    h = jnp.dot(x, W0, preferred_element_type=_F32) + b0
    h = jnp.maximum(_ln(h, g0, p0), 0.0)
    h = jnp.dot(h, W1, preferred_element_type=_F32) + b1
    h = jnp.maximum(_ln(h, g1, p1), 0.0)
    return jnp.dot(h, W2, preferred_element_type=_F32) + b2


def _enc_args(enc):
    r = lambda v: v.reshape(1, -1)
    return [enc["W"][0], r(enc["b"][0]), r(enc["lg"][0]), r(enc["lb"][0]),
            enc["W"][1], r(enc["b"][1]), r(enc["lg"][1]), r(enc["lb"][1]),
            enc["W"][2], r(enc["b"][2])]


def _node_enc_kernel(x, W0, b0, g0, p0, W1, b1, g1, p1, W2, b2, Wms, Wmt,
                     h_out, a_out, b_out):
    h = _mlp3(x[...], W0[...], b0[...], g0[...], p0[...], W1[...], b1[...],
              g1[...], p1[...], W2[...], b2[...])
    h_out[...] = h
    a_out[...] = jnp.dot(h, Wms[...], preferred_element_type=_F32).astype(_BF)
    b_out[...] = jnp.dot(h, Wmt[...], preferred_element_type=_F32).astype(_BF)


def _node_enc(node_feats, enc, Wms0, Wmt0):
    ws = _enc_args(enc) + [Wms0, Wmt0]
    full = [pl.BlockSpec(w.shape, lambda i: (0, 0)) for w in ws]
    return pl.pallas_call(
        _node_enc_kernel,
        grid=(_N // _NB,),
        in_specs=[pl.BlockSpec((_NB, 2), lambda i: (i, 0))] + full,
        out_specs=[pl.BlockSpec((_NB, _H), lambda i: (i, 0))] * 3,
        out_shape=[jax.ShapeDtypeStruct((_N, _H), _F32),
                   jax.ShapeDtypeStruct((_N, _H), _BF),
                   jax.ShapeDtypeStruct((_N, _H), _BF)],
    )(node_feats, *ws)


def _edge_enc_kernel(x, W0, b0, g0, p0, W1, b1, g1, p1, W2, b2,
                     We0, d0, We1, d1, We2, d2, c0, c1, c2, esum):
    i = pl.program_id(0)

    @pl.when(i < _NVB)
    def _():
        h = _mlp3(x[...], W0[...], b0[...], g0[...], p0[...], W1[...],
                  b1[...], g1[...], p1[...], W2[...], b2[...])
        c0[...] = (jnp.dot(h, We0[...], preferred_element_type=_F32)
                   + d0[...]).astype(_BF)
        c1[...] = (jnp.dot(h, We1[...], preferred_element_type=_F32)
                   + d1[...]).astype(_BF)
        c2[...] = (jnp.dot(h, We2[...], preferred_element_type=_F32)
                   + d2[...]).astype(_BF)

        @pl.when(i == 0)
        def _():
            esum[...] = jnp.zeros_like(esum)

        esum[...] += jnp.sum(h.reshape(_EB // 8, 8, _H), axis=0)

    @pl.when(i >= _NVB)
    def _():
        neg = jnp.full((_EB, _H), -1e30, _BF)
        c0[...] = neg
        c1[...] = neg
        c2[...] = neg


def _edge_enc(edge_feats, enc, Wme, bm1):
    ws = _enc_args(enc)
    for l in range(3):
        ws += [Wme[l], bm1[l].reshape(1, _H)]
    full = [pl.BlockSpec(w.shape, lambda i: (0, 0)) for w in ws]
    return pl.pallas_call(
        _edge_enc_kernel,
        grid=(_EPAD // _EB,),
        in_specs=[pl.BlockSpec((_EB, 5), lambda i: (i, 0))] + full,
        out_specs=[pl.BlockSpec((_EB, _H), lambda i: (i, 0))] * 3
                  + [pl.BlockSpec((8, _H), lambda i: (0, 0))],
        out_shape=[jax.ShapeDtypeStruct((_EPAD, _H), _BF)] * 3
                  + [jax.ShapeDtypeStruct((8, _H), _F32)],
    )(edge_feats, *ws)


def _update_mid_kernel(h_ref, s_ref, deg_ref, Wm2, bm2, Wu1a, Wu1b, bu1,
                       Wu2, bu2, WmsN, WmtN, h_out, a_out, b_out):
    h = h_ref[...]
    agg = (jnp.dot(s_ref[...], Wm2[...], preferred_element_type=_F32)
           + deg_ref[...][:, 0:1] * bm2[...])
    pre = (jnp.dot(h, Wu1a[...], preferred_element_type=_F32)
           + jnp.dot(agg, Wu1b[...], preferred_element_type=_F32) + bu1[...])
    h2 = h + jnp.dot(jnp.maximum(pre, 0.0), Wu2[...],
                     preferred_element_type=_F32) + bu2[...]
    h_out[...] = h2
    a_out[...] = jnp.dot(h2, WmsN[...],
                         preferred_element_type=_F32).astype(_BF)
    b_out[...] = jnp.dot(h2, WmtN[...],
                         preferred_element_type=_F32).astype(_BF)


def _update_last_kernel(h_ref, s_ref, deg_ref, Wm2, bm2, Wu1a, Wu1b, bu1,
                        Wu2, bu2, nsum):
    h = h_ref[...]
    agg = (jnp.dot(s_ref[...], Wm2[...], preferred_element_type=_F32)
           + deg_ref[...][:, 0:1] * bm2[...])
    pre = (jnp.dot(h, Wu1a[...], preferred_element_type=_F32)
           + jnp.dot(agg, Wu1b[...], preferred_element_type=_F32) + bu1[...])
    h2 = h + jnp.dot(jnp.maximum(pre, 0.0), Wu2[...],
                     preferred_element_type=_F32) + bu2[...]

    @pl.when(pl.program_id(0) == 0)
    def _():
        nsum[...] = jnp.zeros_like(nsum)

    nsum[...] += jnp.sum(h2.reshape(_NB // 8, 8, _H), axis=0)


def _layer_ws(lp):
    return [lp["Wm2"][jnp.array(_PI)], lp["bm2"].reshape(1, _H),
            lp["Wu1"][0:_H], lp["Wu1"][_H:2 * _H], lp["bu1"].reshape(1, _H),
            lp["Wu2"], lp["bu2"].reshape(1, _H)]


def _update_mid(h, S, deg, lp, WmsN, WmtN):
    ws = _layer_ws(lp) + [WmsN, WmtN]
    full = [pl.BlockSpec(w.shape, lambda i: (0, 0)) for w in ws]
    return pl.pallas_call(
        _update_mid_kernel,
        grid=(_N // _NB,),
        in_specs=[pl.BlockSpec((_NB, _H), lambda i: (i, 0)),
                  pl.BlockSpec((_NB, _H), lambda i: (i, 0)),
                  pl.BlockSpec((_NB, 16), lambda i: (i, 0))] + full,
        out_specs=[pl.BlockSpec((_NB, _H), lambda i: (i, 0))] * 3,
        out_shape=[jax.ShapeDtypeStruct((_N, _H), _F32),
                   jax.ShapeDtypeStruct((_N, _H), _BF),
                   jax.ShapeDtypeStruct((_N, _H), _BF)],
    )(h, S, deg, *ws)


def _update_last(h, S, deg, lp):
    ws = _layer_ws(lp)
    full = [pl.BlockSpec(w.shape, lambda i: (0, 0)) for w in ws]
    return pl.pallas_call(
        _update_last_kernel,
        grid=(_N // _NB,),
        in_specs=[pl.BlockSpec((_NB, _H), lambda i: (i, 0)),
                  pl.BlockSpec((_NB, _H), lambda i: (i, 0)),
                  pl.BlockSpec((_NB, 16), lambda i: (i, 0))] + full,
        out_specs=pl.BlockSpec((8, _H), lambda i: (0, 0)),
        out_shape=jax.ShapeDtypeStruct((8, _H), _F32),
    )(h, S, deg, *ws)


def _head_kernel(ns, es, W0, b0, g0, p0, W1, b1, g1, p1, W2, b2, out):
    nmean = jnp.sum(ns[...], axis=0, keepdims=True) * (1.0 / _N)
    emean = jnp.sum(es[...], axis=0, keepdims=True) * (1.0 / _E)
    g = jnp.concatenate([nmean, emean], axis=1)
    g8 = jnp.concatenate([g, jnp.zeros((7, 2 * _H), _F32)], axis=0)
    h = _mlp3(g8, W0[...], b0[...], g0[...], p0[...], W1[...], b1[...],
              g1[...], p1[...], W2[...], b2[...])
    y = h[0:1, 0:1]
    out[...] = jnp.maximum(y, 0.0) + jnp.log1p(jnp.exp(-jnp.abs(y)))


def _head(nsum, esum, enc):
    ws = _enc_args(enc)
    full = [pl.BlockSpec(w.shape, lambda: (0, 0)) for w in ws]
    return pl.pallas_call(
        _head_kernel,
        in_specs=[pl.BlockSpec((8, _H), lambda: (0, 0)),
                  pl.BlockSpec((8, _H), lambda: (0, 0))] + full,
        out_specs=pl.BlockSpec((1, 1), lambda: (0, 0)),
        out_shape=jax.ShapeDtypeStruct((1, 1), _F32),
    )(nsum, esum, *ws)


def _sc_local_idx(tgt_v, idx_v, half0):
    def cidx(k, _):
        kk = pl.multiple_of(k * 16, 16)
        t16 = tgt_v[pl.ds(kk, 16)]
        loc = t16 - half0
        ok = (loc >= 0) & (loc < _HALF)
        idx_v[pl.ds(kk, 16)] = jnp.where(ok, loc, _TRASH0 + (t16 & 63))
        return 0
    lax.fori_loop(0, _CH // 16, cidx, 0)


def _sc_zero_acc(z_v, acc, s, width):
    zero = jnp.zeros((16,), _F32)

    def zfill(i, _):
        for j in range(width // 16):
            z_v[i, pl.ds(j * 16, 16)] = zero
        return 0
    lax.fori_loop(0, _ZR, zfill, 0)

    def zcp(t, _):
        pltpu.sync_copy(z_v, acc.at[pl.ds(s * _STRIPE + t * _ZR, _ZR)])
        return 0
    lax.fori_loop(0, _STRIPE // _ZR, zcp, 0)


def _sc_copy_out(acc, out_hbm, s, half0):
    row0 = s * _STRIPE

    @pl.when(s < 15)
    def _():
        pltpu.sync_copy(acc.at[pl.ds(row0, _STRIPE)],
                        out_hbm.at[pl.ds(half0 + row0, _STRIPE)])

    @pl.when(s == 15)
    def _():
        pltpu.sync_copy(acc.at[pl.ds(row0, _LAST)],
                        out_hbm.at[pl.ds(half0 + row0, _LAST)])


def _sc_layer_body(a_hbm, b_hbm, c_hbm, src_hbm, tgt_hbm, out_hbm,
                   src_v, tgt_v, idx_v, a_v, b_v, c_v, o_v, z_v, acc,
                   sem_i, sem_g, sem_c, sem_o):
    cc = lax.axis_index("c")
    s = lax.axis_index("s")
    half0 = cc * _HALF
    _sc_zero_acc(z_v, acc, s, _H)
    plsc.subcore_barrier()
    ebase = s * _EPW

    def in_sl(g):
        return pl.ds(ebase + g * _CH, _CH)

    def start_in(g, b):
        pltpu.async_copy(src_hbm.at[in_sl(g)], src_v[b], sem_i[b])
        pltpu.async_copy(tgt_hbm.at[in_sl(g)], tgt_v[b], sem_i[b])

    def wait_in(g, b):
        pltpu.make_async_copy(src_hbm.at[in_sl(g)], src_v[b],
                              sem_i[b]).wait()
        pltpu.make_async_copy(tgt_hbm.at[in_sl(g)], tgt_v[b],
                              sem_i[b]).wait()

    def start_fetch(g, b):
        pltpu.async_copy(a_hbm.at[src_v[b]], a_v[b], sem_g[b])
        pltpu.async_copy(b_hbm.at[tgt_v[b]], b_v[b], sem_g[b])
        pltpu.async_copy(c_hbm.at[in_sl(g)], c_v[b], sem_c[b])

    def wait_fetch(g, b):
        pltpu.make_async_copy(a_hbm.at[src_v[b]], a_v[b], sem_g[b]).wait()
        pltpu.make_async_copy(b_hbm.at[tgt_v[b]], b_v[b], sem_g[b]).wait()
        pltpu.make_async_copy(c_hbm.at[in_sl(g)], c_v[b], sem_c[b]).wait()

    def wait_scat(b):
        pltpu.make_async_copy(o_v[b], acc.at[idx_v[b]], sem_o[b]).wait()

    start_in(0, 0)
    start_in(1, 1)
    wait_in(0, 0)
    _sc_local_idx(tgt_v[0], idx_v[0], half0)
    start_fetch(0, 0)

    @pl.loop(0, _NCH, step=2)
    def _(g0):
        for b in range(2):
            g = g0 + b
            bn = 1 - b
            wait_fetch(g, b)

            @plsc.parallel_loop(0, _CH, step=1)
            def _(r):
                for j in range(2):
                    sl = pl.ds(j * 32, 32)
                    aw = plsc.bitcast(a_v[b][r, sl], jnp.int32)
                    bw = plsc.bitcast(b_v[b][r, sl], jnp.int32)
                    cw = plsc.bitcast(c_v[b][r, sl], jnp.int32)
                    lo = [plsc.bitcast(w << 16, _F32) for w in (aw, bw, cw)]
                    hi = [plsc.bitcast(w & jnp.int32(-65536), _F32)
                          for w in (aw, bw, cw)]
                    o_v[b][r, pl.ds(j * 32, 16)] = jnp.maximum(
                        lo[0] + lo[1] + lo[2], 0.0)
                    o_v[b][r, pl.ds(j * 32 + 16, 16)] = jnp.maximum(
                        hi[0] + hi[1] + hi[2], 0.0)

            pltpu.async_copy(o_v[b], acc.at[idx_v[b]], sem_o[b], add=True)

            @pl.when(g + 1 < _NCH)
            def _():
                wait_in(g + 1, bn)
                if b == 0:
                    @pl.when(g0 >= 2)
                    def _():
                        wait_scat(bn)
                else:
                    wait_scat(bn)
                _sc_local_idx(tgt_v[bn], idx_v[bn], half0)
                start_fetch(g + 1, bn)

            @pl.when(g + 2 < _NCH)
            def _():
                start_in(g + 2, b)

    wait_scat(0)
    wait_scat(1)
    plsc.subcore_barrier()
    _sc_copy_out(acc, out_hbm, s, half0)


def _sc_layer(A, B, C, src, tgt):
    mesh = plsc.VectorSubcoreMesh(core_axis_name="c", subcore_axis_name="s")
    fn = pl.kernel(
        _sc_layer_body,
        mesh=mesh,
        compiler_params=pltpu.CompilerParams(use_tc_tiling_on_sc=False,
                                             needs_layout_passes=False),
        out_type=jax.ShapeDtypeStruct((_N, _H), _F32),
        scratch_types=[
            [pltpu.VMEM((_CH,), jnp.int32)] * 2,
            [pltpu.VMEM((_CH,), jnp.int32)] * 2,
            [pltpu.VMEM((_CH,), jnp.int32)] * 2,
            [pltpu.VMEM((_CH, _H), _BF)] * 2,
            [pltpu.VMEM((_CH, _H), _BF)] * 2,
            [pltpu.VMEM((_CH, _H), _BF)] * 2,
            [pltpu.VMEM((_CH, _H), _F32)] * 2,
            pltpu.VMEM((_ZR, _H), _F32),
            pltpu.VMEM_SHARED((_HPAD, _H), _F32),
            [pltpu.SemaphoreType.DMA] * 2,
            [pltpu.SemaphoreType.DMA] * 2,
            [pltpu.SemaphoreType.DMA] * 2,
            [pltpu.SemaphoreType.DMA] * 2,
        ])
    return fn(A, B, C, src, tgt)


def _sc_deg_body(tgt_hbm, out_hbm, tgt_v, idx_v, val_v, z_v, acc,
                 sem_i, sem_o):
    cc = lax.axis_index("c")
    s = lax.axis_index("s")
    half0 = cc * _HALF
    _sc_zero_acc(z_v, acc, s, 16)
    one_hot = jnp.where(lax.iota(jnp.int32, 16) == 0,
                        jnp.float32(1.0), jnp.float32(0.0))

    def vfill(i, _):
        val_v[i, pl.ds(0, 16)] = one_hot
        return 0
    lax.fori_loop(0, _CH, vfill, 0)
    plsc.subcore_barrier()
    ebase = s * _EPW

    for b in range(2):
        pltpu.async_copy(tgt_hbm.at[pl.ds(ebase + b * _CH, _CH)], tgt_v[b],
                         sem_i[b])

    @pl.loop(0, _NCH, step=2)
    def _(g0):
        for b in range(2):
            g = g0 + b
            off = ebase + g * _CH
            pltpu.make_async_copy(tgt_hbm.at[pl.ds(off, _CH)], tgt_v[b],
                                  sem_i[b]).wait()

            @pl.when(g0 >= 2)
            def _():
                pltpu.make_async_copy(val_v, acc.at[idx_v[b]],
                                      sem_o[b]).wait()

            _sc_local_idx(tgt_v[b], idx_v[b], half0)
            pltpu.async_copy(val_v, acc.at[idx_v[b]], sem_o[b], add=True)

            @pl.when(g + 2 < _NCH)
            def _():
                pltpu.async_copy(tgt_hbm.at[pl.ds(off + 2 * _CH, _CH)],
                                 tgt_v[b], sem_i[b])

    for b in range(2):
        pltpu.make_async_copy(val_v, acc.at[idx_v[b]], sem_o[b]).wait()
    plsc.subcore_barrier()
    _sc_copy_out(acc, out_hbm, s, half0)


def _sc_deg(tgt):
    mesh = plsc.VectorSubcoreMesh(core_axis_name="c", subcore_axis_name="s")
    fn = pl.kernel(
        _sc_deg_body,
        mesh=mesh,
        compiler_params=pltpu.CompilerParams(use_tc_tiling_on_sc=False),
        out_type=jax.ShapeDtypeStruct((_N, 16), _F32),
        scratch_types=[
            [pltpu.VMEM((_CH,), jnp.int32)] * 2,
            [pltpu.VMEM((_CH,), jnp.int32)] * 2,
            pltpu.VMEM((_CH, 16), _F32),
            pltpu.VMEM((_ZR, 16), _F32),
            pltpu.VMEM_SHARED((_HPAD, 16), _F32),
            [pltpu.SemaphoreType.DMA] * 2,
            [pltpu.SemaphoreType.DMA] * 2,
        ])
    return fn(tgt)


def kernel(node_feats, edge_feats, params, edge_index):
    layers = params["layers"]
    npad = _EPAD - _E
    src = jnp.concatenate([edge_index[0], jnp.zeros((npad,), jnp.int32)])
    tgt_g = jnp.concatenate([edge_index[1], jnp.zeros((npad,), jnp.int32)])
    tgt_i = jnp.concatenate([edge_index[1],
                             jnp.full((npad,), -1, jnp.int32)])
    ef_pad = jnp.concatenate(
        [edge_feats, jnp.zeros((npad, 5), edge_feats.dtype)])
    Wms = [lp["Wm1"][0:_H] for lp in layers]
    Wmt = [lp["Wm1"][_H:2 * _H] for lp in layers]
    Wme = [lp["Wm1"][2 * _H:3 * _H] for lp in layers]
    bm1 = [lp["bm1"] for lp in layers]

    h, A, B = _node_enc(node_feats, params["node_enc"], Wms[0], Wmt[0])
    C0, C1, C2, esum = _edge_enc(ef_pad, params["edge_enc"], Wme, bm1)
    Cs = [C0, C1, C2]
    deg = _sc_deg(tgt_i)

    for l in range(3):
        S = _sc_layer(A, B, Cs[l], src, tgt_g)
        if l < 2:
            h, A, B = _update_mid(h, S, deg, layers[l], Wms[l + 1],
                                  Wmt[l + 1])
        else:
            nsum = _update_last(h, S, deg, layers[l])
    return _head(nsum, esum, params["head"])


# speedup vs baseline: 1.1141x; 1.1141x over previous
import functools

import jax
import jax.numpy as jnp
from jax import lax
from jax.experimental import pallas as pl
from jax.experimental.pallas import tpu as pltpu
from jax.experimental.pallas import tpu_sc as plsc

_N = 50000
_E = 800000
_H = 64
_NB = 2000
_EB = 3200
_HALF = _N // 2
_HPAD = 25088
_STRIPE = _HPAD // 16
_LAST = _HALF - 15 * _STRIPE
_TRASH0 = 25008
_CH = 64
_EPAD = 819200
_NVB = _E // _EB
_EPW = _EPAD // 16
_NCH = _EPW // _CH
_ZR = 49
_EPS = 1e-5
_F32 = jnp.float32
_BF = jnp.bfloat16
_PI = ([2 * i for i in range(16)] + [2 * i + 1 for i in range(16)]
       + [32 + 2 * i for i in range(16)] + [33 + 2 * i for i in range(16)])


def _ln(h, g, b):
    mu = jnp.mean(h, axis=-1, keepdims=True)
    d = h - mu
    var = jnp.mean(d * d, axis=-1, keepdims=True)
    return d * lax.rsqrt(var + _EPS) * g + b


def _mlp3(x, W0, b0, g0, p0, W1, b1, g1, p1, W2, b2):
    h = jnp.dot(x, W0, preferred_element_type=_F32) + b0
    h = jnp.maximum(_ln(h, g0, p0), 0.0)
    h = jnp.dot(h, W1, preferred_element_type=_F32) + b1
    h = jnp.maximum(_ln(h, g1, p1), 0.0)
    return jnp.dot(h, W2, preferred_element_type=_F32) + b2


def _enc_args(enc):
    r = lambda v: v.reshape(1, -1)
    return [enc["W"][0], r(enc["b"][0]), r(enc["lg"][0]), r(enc["lb"][0]),
            enc["W"][1], r(enc["b"][1]), r(enc["lg"][1]), r(enc["lb"][1]),
            enc["W"][2], r(enc["b"][2])]


def _node_enc_kernel(x, W0, b0, g0, p0, W1, b1, g1, p1, W2, b2, Wms, Wmt,
                     h_out, a_out, b_out):
    h = _mlp3(x[...], W0[...], b0[...], g0[...], p0[...], W1[...], b1[...],
              g1[...], p1[...], W2[...], b2[...])
    h_out[...] = h
    a_out[...] = jnp.dot(h, Wms[...], preferred_element_type=_F32).astype(_BF)
    b_out[...] = jnp.dot(h, Wmt[...], preferred_element_type=_F32).astype(_BF)


def _node_enc(node_feats, enc, Wms0, Wmt0):
    ws = _enc_args(enc) + [Wms0, Wmt0]
    full = [pl.BlockSpec(w.shape, lambda i: (0, 0)) for w in ws]
    return pl.pallas_call(
        _node_enc_kernel,
        grid=(_N // _NB,),
        in_specs=[pl.BlockSpec((_NB, 2), lambda i: (i, 0))] + full,
        out_specs=[pl.BlockSpec((_NB, _H), lambda i: (i, 0))] * 3,
        out_shape=[jax.ShapeDtypeStruct((_N, _H), _F32),
                   jax.ShapeDtypeStruct((_N, _H), _BF),
                   jax.ShapeDtypeStruct((_N, _H), _BF)],
    )(node_feats, *ws)


def _edge_enc_kernel(x, W0, b0, g0, p0, W1, b1, g1, p1, W2, b2,
                     We0, d0, We1, d1, We2, d2, c0, c1, c2, esum):
    i = pl.program_id(0)

    @pl.when(i < _NVB)
    def _():
        h = _mlp3(x[...], W0[...], b0[...], g0[...], p0[...], W1[...],
                  b1[...], g1[...], p1[...], W2[...], b2[...])
        c0[...] = (jnp.dot(h, We0[...], preferred_element_type=_F32)
                   + d0[...]).astype(_BF)
        c1[...] = (jnp.dot(h, We1[...], preferred_element_type=_F32)
                   + d1[...]).astype(_BF)
        c2[...] = (jnp.dot(h, We2[...], preferred_element_type=_F32)
                   + d2[...]).astype(_BF)

        @pl.when(i == 0)
        def _():
            esum[...] = jnp.zeros_like(esum)

        esum[...] += jnp.sum(h.reshape(_EB // 8, 8, _H), axis=0)

    @pl.when(i >= _NVB)
    def _():
        neg = jnp.full((_EB, _H), -1e30, _BF)
        c0[...] = neg
        c1[...] = neg
        c2[...] = neg


def _edge_enc(edge_feats, enc, Wme, bm1):
    ws = _enc_args(enc)
    for l in range(3):
        ws += [Wme[l], bm1[l].reshape(1, _H)]
    full = [pl.BlockSpec(w.shape, lambda i: (0, 0)) for w in ws]
    return pl.pallas_call(
        _edge_enc_kernel,
        grid=(_EPAD // _EB,),
        in_specs=[pl.BlockSpec((_EB, 5), lambda i: (i, 0))] + full,
        out_specs=[pl.BlockSpec((_EB, _H), lambda i: (i, 0))] * 3
                  + [pl.BlockSpec((8, _H), lambda i: (0, 0))],
        out_shape=[jax.ShapeDtypeStruct((_EPAD, _H), _BF)] * 3
                  + [jax.ShapeDtypeStruct((8, _H), _F32)],
    )(edge_feats, *ws)


def _update_mid_kernel(h_ref, s_ref, deg_ref, Wm2, bm2, Wu1a, Wu1b, bu1,
                       Wu2, bu2, WmsN, WmtN, h_out, a_out, b_out):
    h = h_ref[...]
    agg = (jnp.dot(s_ref[...], Wm2[...], preferred_element_type=_F32)
           + deg_ref[...][:, 0:1] * bm2[...])
    pre = (jnp.dot(h, Wu1a[...], preferred_element_type=_F32)
           + jnp.dot(agg, Wu1b[...], preferred_element_type=_F32) + bu1[...])
    h2 = h + jnp.dot(jnp.maximum(pre, 0.0), Wu2[...],
                     preferred_element_type=_F32) + bu2[...]
    h_out[...] = h2
    a_out[...] = jnp.dot(h2, WmsN[...],
                         preferred_element_type=_F32).astype(_BF)
    b_out[...] = jnp.dot(h2, WmtN[...],
                         preferred_element_type=_F32).astype(_BF)


def _update_last_kernel(h_ref, s_ref, deg_ref, Wm2, bm2, Wu1a, Wu1b, bu1,
                        Wu2, bu2, nsum):
    h = h_ref[...]
    agg = (jnp.dot(s_ref[...], Wm2[...], preferred_element_type=_F32)
           + deg_ref[...][:, 0:1] * bm2[...])
    pre = (jnp.dot(h, Wu1a[...], preferred_element_type=_F32)
           + jnp.dot(agg, Wu1b[...], preferred_element_type=_F32) + bu1[...])
    h2 = h + jnp.dot(jnp.maximum(pre, 0.0), Wu2[...],
                     preferred_element_type=_F32) + bu2[...]

    @pl.when(pl.program_id(0) == 0)
    def _():
        nsum[...] = jnp.zeros_like(nsum)

    nsum[...] += jnp.sum(h2.reshape(_NB // 8, 8, _H), axis=0)


def _layer_ws(lp):
    return [lp["Wm2"][jnp.array(_PI)], lp["bm2"].reshape(1, _H),
            lp["Wu1"][0:_H], lp["Wu1"][_H:2 * _H], lp["bu1"].reshape(1, _H),
            lp["Wu2"], lp["bu2"].reshape(1, _H)]


def _update_mid(h, S, deg, lp, WmsN, WmtN):
    ws = _layer_ws(lp) + [WmsN, WmtN]
    full = [pl.BlockSpec(w.shape, lambda i: (0, 0)) for w in ws]
    return pl.pallas_call(
        _update_mid_kernel,
        grid=(_N // _NB,),
        in_specs=[pl.BlockSpec((_NB, _H), lambda i: (i, 0)),
                  pl.BlockSpec((_NB, _H), lambda i: (i, 0)),
                  pl.BlockSpec((_NB, 16), lambda i: (i, 0))] + full,
        out_specs=[pl.BlockSpec((_NB, _H), lambda i: (i, 0))] * 3,
        out_shape=[jax.ShapeDtypeStruct((_N, _H), _F32),
                   jax.ShapeDtypeStruct((_N, _H), _BF),
                   jax.ShapeDtypeStruct((_N, _H), _BF)],
    )(h, S, deg, *ws)


def _update_last(h, S, deg, lp):
    ws = _layer_ws(lp)
    full = [pl.BlockSpec(w.shape, lambda i: (0, 0)) for w in ws]
    return pl.pallas_call(
        _update_last_kernel,
        grid=(_N // _NB,),
        in_specs=[pl.BlockSpec((_NB, _H), lambda i: (i, 0)),
                  pl.BlockSpec((_NB, _H), lambda i: (i, 0)),
                  pl.BlockSpec((_NB, 16), lambda i: (i, 0))] + full,
        out_specs=pl.BlockSpec((8, _H), lambda i: (0, 0)),
        out_shape=jax.ShapeDtypeStruct((8, _H), _F32),
    )(h, S, deg, *ws)


def _head_kernel(ns, es, W0, b0, g0, p0, W1, b1, g1, p1, W2, b2, out):
    nmean = jnp.sum(ns[...], axis=0, keepdims=True) * (1.0 / _N)
    emean = jnp.sum(es[...], axis=0, keepdims=True) * (1.0 / _E)
    g = jnp.concatenate([nmean, emean], axis=1)
    g8 = jnp.concatenate([g, jnp.zeros((7, 2 * _H), _F32)], axis=0)
    h = _mlp3(g8, W0[...], b0[...], g0[...], p0[...], W1[...], b1[...],
              g1[...], p1[...], W2[...], b2[...])
    y = h[0:1, 0:1]
    out[...] = jnp.maximum(y, 0.0) + jnp.log1p(jnp.exp(-jnp.abs(y)))


def _head(nsum, esum, enc):
    ws = _enc_args(enc)
    full = [pl.BlockSpec(w.shape, lambda: (0, 0)) for w in ws]
    return pl.pallas_call(
        _head_kernel,
        in_specs=[pl.BlockSpec((8, _H), lambda: (0, 0)),
                  pl.BlockSpec((8, _H), lambda: (0, 0))] + full,
        out_specs=pl.BlockSpec((1, 1), lambda: (0, 0)),
        out_shape=jax.ShapeDtypeStruct((1, 1), _F32),
    )(nsum, esum, *ws)


def _sc_local_idx(tgt_v, idx_v, half0):
    def cidx(k, _):
        kk = pl.multiple_of(k * 16, 16)
        t16 = tgt_v[pl.ds(kk, 16)]
        loc = t16 - half0
        ok = (loc >= 0) & (loc < _HALF)
        idx_v[pl.ds(kk, 16)] = jnp.where(ok, loc, _TRASH0 + (t16 & 63))
        return 0
    lax.fori_loop(0, _CH // 16, cidx, 0)


def _sc_zero_acc(z_v, acc, s, width):
    zero = jnp.zeros((16,), _F32)

    def zfill(i, _):
        for j in range(width // 16):
            z_v[i, pl.ds(j * 16, 16)] = zero
        return 0
    lax.fori_loop(0, _ZR, zfill, 0)

    def zcp(t, _):
        pltpu.sync_copy(z_v, acc.at[pl.ds(s * _STRIPE + t * _ZR, _ZR)])
        return 0
    lax.fori_loop(0, _STRIPE // _ZR, zcp, 0)


def _sc_copy_out(acc, out_hbm, s, half0):
    row0 = s * _STRIPE

    @pl.when(s < 15)
    def _():
        pltpu.sync_copy(acc.at[pl.ds(row0, _STRIPE)],
                        out_hbm.at[pl.ds(half0 + row0, _STRIPE)])

    @pl.when(s == 15)
    def _():
        pltpu.sync_copy(acc.at[pl.ds(row0, _LAST)],
                        out_hbm.at[pl.ds(half0 + row0, _LAST)])


def _sc_layer_body(a_hbm, b_hbm, c_hbm, src_hbm, tgt_hbm, out_hbm,
                   src_v, tgt_v, idx_v, a_v, b_v, c_v, o_v, z_v, acc,
                   sem_i, sem_g, sem_c, sem_o):
    cc = lax.axis_index("c")
    s = lax.axis_index("s")
    half0 = cc * _HALF
    _sc_zero_acc(z_v, acc, s, _H)
    plsc.subcore_barrier()
    ebase = s * _EPW

    def in_sl(g):
        return pl.ds(ebase + g * _CH, _CH)

    def start_in(g, b):
        pltpu.async_copy(src_hbm.at[in_sl(g)], src_v[b], sem_i[b])
        pltpu.async_copy(tgt_hbm.at[in_sl(g)], tgt_v[b], sem_i[b])

    def wait_in(g, b):
        pltpu.make_async_copy(src_hbm.at[in_sl(g)], src_v[b],
                              sem_i[b]).wait()
        pltpu.make_async_copy(tgt_hbm.at[in_sl(g)], tgt_v[b],
                              sem_i[b]).wait()

    def start_fetch(g, b):
        pltpu.async_copy(a_hbm.at[src_v[b]], a_v[b], sem_g[b])
        pltpu.async_copy(b_hbm.at[tgt_v[b]], b_v[b], sem_g[b])
        pltpu.async_copy(c_hbm.at[in_sl(g)], c_v[b], sem_c[b])

    def wait_fetch(g, b):
        pltpu.make_async_copy(a_hbm.at[src_v[b]], a_v[b], sem_g[b]).wait()
        pltpu.make_async_copy(b_hbm.at[tgt_v[b]], b_v[b], sem_g[b]).wait()
        pltpu.make_async_copy(c_hbm.at[in_sl(g)], c_v[b], sem_c[b]).wait()

    def wait_scat(b):
        pltpu.make_async_copy(o_v[b], acc.at[idx_v[b]], sem_o[b]).wait()

    start_in(0, 0)
    start_in(1, 1)
    wait_in(0, 0)
    _sc_local_idx(tgt_v[0], idx_v[0], half0)
    start_fetch(0, 0)

    @pl.loop(0, _NCH, step=2)
    def _(g0):
        for b in range(2):
            g = g0 + b
            bn = 1 - b
            wait_fetch(g, b)

            @plsc.parallel_loop(0, _CH, step=1)
            def _(r):
                for j in range(2):
                    sl = pl.ds(j * 32, 32)
                    aw = plsc.bitcast(a_v[b][r, sl], jnp.int32)
                    bw = plsc.bitcast(b_v[b][r, sl], jnp.int32)
                    cw = plsc.bitcast(c_v[b][r, sl], jnp.int32)
                    lo = [plsc.bitcast(w << 16, _F32) for w in (aw, bw, cw)]
                    hi = [plsc.bitcast(w & jnp.int32(-65536), _F32)
                          for w in (aw, bw, cw)]
                    o_v[b][r, pl.ds(j * 32, 16)] = jnp.maximum(
                        lo[0] + lo[1] + lo[2], 0.0)
                    o_v[b][r, pl.ds(j * 32 + 16, 16)] = jnp.maximum(
                        hi[0] + hi[1] + hi[2], 0.0)

            pltpu.async_copy(o_v[b], acc.at[idx_v[b]], sem_o[b], add=True)

            @pl.when(g + 1 < _NCH)
            def _():
                wait_in(g + 1, bn)
                if b == 0:
                    @pl.when(g0 >= 2)
                    def _():
                        wait_scat(bn)
                else:
                    wait_scat(bn)
                _sc_local_idx(tgt_v[bn], idx_v[bn], half0)
                start_fetch(g + 1, bn)

            @pl.when(g + 2 < _NCH)
            def _():
                start_in(g + 2, b)

    wait_scat(0)
    wait_scat(1)
    plsc.subcore_barrier()
    _sc_copy_out(acc, out_hbm, s, half0)


def _sc_layer(A, B, C, src, tgt):
    mesh = plsc.VectorSubcoreMesh(core_axis_name="c", subcore_axis_name="s")
    fn = pl.kernel(
        _sc_layer_body,
        mesh=mesh,
        compiler_params=pltpu.CompilerParams(use_tc_tiling_on_sc=False,
                                             needs_layout_passes=False),
        out_type=jax.ShapeDtypeStruct((_N, _H), _F32),
        scratch_types=[
            [pltpu.VMEM((_CH,), jnp.int32)] * 2,
            [pltpu.VMEM((_CH,), jnp.int32)] * 2,
            [pltpu.VMEM((_CH,), jnp.int32)] * 2,
            [pltpu.VMEM((_CH, _H), _BF)] * 2,
            [pltpu.VMEM((_CH, _H), _BF)] * 2,
            [pltpu.VMEM((_CH, _H), _BF)] * 2,
            [pltpu.VMEM((_CH, _H), _F32)] * 2,
            pltpu.VMEM((_ZR, _H), _F32),
            pltpu.VMEM_SHARED((_HPAD, _H), _F32),
            [pltpu.SemaphoreType.DMA] * 2,
            [pltpu.SemaphoreType.DMA] * 2,
            [pltpu.SemaphoreType.DMA] * 2,
            [pltpu.SemaphoreType.DMA] * 2,
        ])
    return fn(A, B, C, src, tgt)


def _sc_deg_body(tgt_hbm, out_hbm, tgt_v, idx_v, val_v, z_v, acc,
                 sem_i, sem_o):
    cc = lax.axis_index("c")
    s = lax.axis_index("s")
    half0 = cc * _HALF
    _sc_zero_acc(z_v, acc, s, 16)
    one_hot = jnp.where(lax.iota(jnp.int32, 16) == 0,
                        jnp.float32(1.0), jnp.float32(0.0))

    def vfill(i, _):
        val_v[i, pl.ds(0, 16)] = one_hot
        return 0
    lax.fori_loop(0, _CH, vfill, 0)
    plsc.subcore_barrier()
    ebase = s * _EPW

    for b in range(2):
        pltpu.async_copy(tgt_hbm.at[pl.ds(ebase + b * _CH, _CH)], tgt_v[b],
                         sem_i[b])

    @pl.loop(0, _NCH, step=2)
    def _(g0):
        for b in range(2):
            g = g0 + b
            off = ebase + g * _CH
            pltpu.make_async_copy(tgt_hbm.at[pl.ds(off, _CH)], tgt_v[b],
                                  sem_i[b]).wait()

            @pl.when(g0 >= 2)
            def _():
                pltpu.make_async_copy(val_v, acc.at[idx_v[b]],
                                      sem_o[b]).wait()

            _sc_local_idx(tgt_v[b], idx_v[b], half0)
            pltpu.async_copy(val_v, acc.at[idx_v[b]], sem_o[b], add=True)

            @pl.when(g + 2 < _NCH)
            def _():
                pltpu.async_copy(tgt_hbm.at[pl.ds(off + 2 * _CH, _CH)],
                                 tgt_v[b], sem_i[b])

    for b in range(2):
        pltpu.make_async_copy(val_v, acc.at[idx_v[b]], sem_o[b]).wait()
    plsc.subcore_barrier()
    _sc_copy_out(acc, out_hbm, s, half0)


def _sc_deg(tgt):
    mesh = plsc.VectorSubcoreMesh(core_axis_name="c", subcore_axis_name="s")
    fn = pl.kernel(
        _sc_deg_body,
        mesh=mesh,
        compiler_params=pltpu.CompilerParams(use_tc_tiling_on_sc=False),
        out_type=jax.ShapeDtypeStruct((_N, 16), _F32),
        scratch_types=[
            [pltpu.VMEM((_CH,), jnp.int32)] * 2,
            [pltpu.VMEM((_CH,), jnp.int32)] * 2,
            pltpu.VMEM((_CH, 16), _F32),
            pltpu.VMEM((_ZR, 16), _F32),
            pltpu.VMEM_SHARED((_HPAD, 16), _F32),
            [pltpu.SemaphoreType.DMA] * 2,
            [pltpu.SemaphoreType.DMA] * 2,
        ])
    return fn(tgt)


def kernel(node_feats, edge_feats, params, edge_index):
    layers = params["layers"]
    npad = _EPAD - _E
    src = jnp.concatenate([edge_index[0], jnp.zeros((npad,), jnp.int32)])
    tgt_g = jnp.concatenate([edge_index[1], jnp.zeros((npad,), jnp.int32)])
    tgt_i = jnp.concatenate([edge_index[1],
                             jnp.full((npad,), -1, jnp.int32)])
    ef_pad = jnp.concatenate(
        [edge_feats, jnp.zeros((npad, 5), edge_feats.dtype)])
    Wms = [lp["Wm1"][0:_H] for lp in layers]
    Wmt = [lp["Wm1"][_H:2 * _H] for lp in layers]
    Wme = [lp["Wm1"][2 * _H:3 * _H] for lp in layers]
    bm1 = [lp["bm1"] for lp in layers]

    h, A, B = _node_enc(node_feats, params["node_enc"], Wms[0], Wmt[0])
    C0, C1, C2, esum = _edge_enc(ef_pad, params["edge_enc"], Wme, bm1)
    Cs = [C0, C1, C2]
    deg = _sc_deg(tgt_i)

    for l in range(3):
        S = _sc_layer(A, B, Cs[l], src, tgt_g)
        if l < 2:
            h, A, B = _update_mid(h, S, deg, layers[l], Wms[l + 1],
                                  Wmt[l + 1])
        else:
            nsum = _update_last(h, S, deg, layers[l])
    return _head(nsum, esum, params["head"])


# speedup vs baseline: 1.1245x; 1.0094x over previous
import functools

import jax
import jax.numpy as jnp
from jax import lax
from jax.experimental import pallas as pl
from jax.experimental.pallas import tpu as pltpu
from jax.experimental.pallas import tpu_sc as plsc

_N = 50000
_E = 800000
_H = 64
_NB = 5000
_EB = 6400
_HALF = _N // 2
_HPAD = 25088
_STRIPE = _HPAD // 16
_LAST = _HALF - 15 * _STRIPE
_TRASH0 = 25008
_CH = 64
_EPAD = 819200
_NVB = _E // _EB
_EPW = _EPAD // 16
_NCH = _EPW // _CH
_ZR = 49
_EPS = 1e-5
_F32 = jnp.float32
_BF = jnp.bfloat16
_PI = ([2 * i for i in range(16)] + [2 * i + 1 for i in range(16)]
       + [32 + 2 * i for i in range(16)] + [33 + 2 * i for i in range(16)])


def _ln(h, g, b):
    mu = jnp.mean(h, axis=-1, keepdims=True)
    d = h - mu
    var = jnp.mean(d * d, axis=-1, keepdims=True)
    return d * lax.rsqrt(var + _EPS) * g + b


def _mlp3(x, W0, b0, g0, p0, W1, b1, g1, p1, W2, b2):
    h = jnp.dot(x, W0, preferred_element_type=_F32) + b0
    h = jnp.maximum(_ln(h, g0, p0), 0.0)
    h = jnp.dot(h, W1, preferred_element_type=_F32) + b1
    h = jnp.maximum(_ln(h, g1, p1), 0.0)
    return jnp.dot(h, W2, preferred_element_type=_F32) + b2


def _enc_args(enc):
    r = lambda v: v.reshape(1, -1)
    return [enc["W"][0], r(enc["b"][0]), r(enc["lg"][0]), r(enc["lb"][0]),
            enc["W"][1], r(enc["b"][1]), r(enc["lg"][1]), r(enc["lb"][1]),
            enc["W"][2], r(enc["b"][2])]


def _node_enc_kernel(x, W0, b0, g0, p0, W1, b1, g1, p1, W2, b2, Wms, Wmt,
                     h_out, a_out, b_out):
    h = _mlp3(x[...], W0[...], b0[...], g0[...], p0[...], W1[...], b1[...],
              g1[...], p1[...], W2[...], b2[...])
    h_out[...] = h
    a_out[...] = jnp.dot(h, Wms[...], preferred_element_type=_F32).astype(_BF)
    b_out[...] = jnp.dot(h, Wmt[...], preferred_element_type=_F32).astype(_BF)


def _node_enc(node_feats, enc, Wms0, Wmt0):
    ws = _enc_args(enc) + [Wms0, Wmt0]
    full = [pl.BlockSpec(w.shape, lambda i: (0, 0)) for w in ws]
    return pl.pallas_call(
        _node_enc_kernel,
        grid=(_N // _NB,),
        in_specs=[pl.BlockSpec((_NB, 2), lambda i: (i, 0))] + full,
        out_specs=[pl.BlockSpec((_NB, _H), lambda i: (i, 0))] * 3,
        out_shape=[jax.ShapeDtypeStruct((_N, _H), _F32),
                   jax.ShapeDtypeStruct((_N, _H), _BF),
                   jax.ShapeDtypeStruct((_N, _H), _BF)],
    )(node_feats, *ws)


def _edge_enc_kernel(x, W0, b0, g0, p0, W1, b1, g1, p1, W2, b2,
                     We0, d0, We1, d1, We2, d2, c0, c1, c2, esum):
    i = pl.program_id(0)

    @pl.when(i < _NVB)
    def _():
        h = _mlp3(x[...], W0[...], b0[...], g0[...], p0[...], W1[...],
                  b1[...], g1[...], p1[...], W2[...], b2[...])
        c0[...] = (jnp.dot(h, We0[...], preferred_element_type=_F32)
                   + d0[...]).astype(_BF)
        c1[...] = (jnp.dot(h, We1[...], preferred_element_type=_F32)
                   + d1[...]).astype(_BF)
        c2[...] = (jnp.dot(h, We2[...], preferred_element_type=_F32)
                   + d2[...]).astype(_BF)

        @pl.when(i == 0)
        def _():
            esum[...] = jnp.zeros_like(esum)

        esum[...] += jnp.sum(h.reshape(_EB // 8, 8, _H), axis=0)

    @pl.when(i >= _NVB)
    def _():
        neg = jnp.full((_EB, _H), -1e30, _BF)
        c0[...] = neg
        c1[...] = neg
        c2[...] = neg


def _edge_enc(edge_feats, enc, Wme, bm1):
    ws = _enc_args(enc)
    for l in range(3):
        ws += [Wme[l], bm1[l].reshape(1, _H)]
    full = [pl.BlockSpec(w.shape, lambda i: (0, 0)) for w in ws]
    return pl.pallas_call(
        _edge_enc_kernel,
        grid=(_EPAD // _EB,),
        in_specs=[pl.BlockSpec((_EB, 5), lambda i: (i, 0))] + full,
        out_specs=[pl.BlockSpec((_EB, _H), lambda i: (i, 0))] * 3
                  + [pl.BlockSpec((8, _H), lambda i: (0, 0))],
        out_shape=[jax.ShapeDtypeStruct((_EPAD, _H), _BF)] * 3
                  + [jax.ShapeDtypeStruct((8, _H), _F32)],
    )(edge_feats, *ws)


def _update_mid_kernel(h_ref, s_ref, deg_ref, Wm2, bm2, Wu1a, Wu1b, bu1,
                       Wu2, bu2, WmsN, WmtN, h_out, a_out, b_out):
    h = h_ref[...]
    agg = (jnp.dot(s_ref[...], Wm2[...], preferred_element_type=_F32)
           + deg_ref[...][:, 0:1] * bm2[...])
    pre = (jnp.dot(h, Wu1a[...], preferred_element_type=_F32)
           + jnp.dot(agg, Wu1b[...], preferred_element_type=_F32) + bu1[...])
    h2 = h + jnp.dot(jnp.maximum(pre, 0.0), Wu2[...],
                     preferred_element_type=_F32) + bu2[...]
    h_out[...] = h2
    a_out[...] = jnp.dot(h2, WmsN[...],
                         preferred_element_type=_F32).astype(_BF)
    b_out[...] = jnp.dot(h2, WmtN[...],
                         preferred_element_type=_F32).astype(_BF)


def _update_last_kernel(h_ref, s_ref, deg_ref, Wm2, bm2, Wu1a, Wu1b, bu1,
                        Wu2, bu2, nsum):
    h = h_ref[...]
    agg = (jnp.dot(s_ref[...], Wm2[...], preferred_element_type=_F32)
           + deg_ref[...][:, 0:1] * bm2[...])
    pre = (jnp.dot(h, Wu1a[...], preferred_element_type=_F32)
           + jnp.dot(agg, Wu1b[...], preferred_element_type=_F32) + bu1[...])
    h2 = h + jnp.dot(jnp.maximum(pre, 0.0), Wu2[...],
                     preferred_element_type=_F32) + bu2[...]

    @pl.when(pl.program_id(0) == 0)
    def _():
        nsum[...] = jnp.zeros_like(nsum)

    nsum[...] += jnp.sum(h2.reshape(_NB // 8, 8, _H), axis=0)


def _layer_ws(lp):
    return [lp["Wm2"][jnp.array(_PI)], lp["bm2"].reshape(1, _H),
            lp["Wu1"][0:_H], lp["Wu1"][_H:2 * _H], lp["bu1"].reshape(1, _H),
            lp["Wu2"], lp["bu2"].reshape(1, _H)]


def _update_mid(h, S, deg, lp, WmsN, WmtN):
    ws = _layer_ws(lp) + [WmsN, WmtN]
    full = [pl.BlockSpec(w.shape, lambda i: (0, 0)) for w in ws]
    return pl.pallas_call(
        _update_mid_kernel,
        grid=(_N // _NB,),
        in_specs=[pl.BlockSpec((_NB, _H), lambda i: (i, 0)),
                  pl.BlockSpec((_NB, _H), lambda i: (i, 0)),
                  pl.BlockSpec((_NB, 16), lambda i: (i, 0))] + full,
        out_specs=[pl.BlockSpec((_NB, _H), lambda i: (i, 0))] * 3,
        out_shape=[jax.ShapeDtypeStruct((_N, _H), _F32),
                   jax.ShapeDtypeStruct((_N, _H), _BF),
                   jax.ShapeDtypeStruct((_N, _H), _BF)],
    )(h, S, deg, *ws)


def _update_last(h, S, deg, lp):
    ws = _layer_ws(lp)
    full = [pl.BlockSpec(w.shape, lambda i: (0, 0)) for w in ws]
    return pl.pallas_call(
        _update_last_kernel,
        grid=(_N // _NB,),
        in_specs=[pl.BlockSpec((_NB, _H), lambda i: (i, 0)),
                  pl.BlockSpec((_NB, _H), lambda i: (i, 0)),
                  pl.BlockSpec((_NB, 16), lambda i: (i, 0))] + full,
        out_specs=pl.BlockSpec((8, _H), lambda i: (0, 0)),
        out_shape=jax.ShapeDtypeStruct((8, _H), _F32),
    )(h, S, deg, *ws)


def _head_kernel(ns, es, W0, b0, g0, p0, W1, b1, g1, p1, W2, b2, out):
    nmean = jnp.sum(ns[...], axis=0, keepdims=True) * (1.0 / _N)
    emean = jnp.sum(es[...], axis=0, keepdims=True) * (1.0 / _E)
    g = jnp.concatenate([nmean, emean], axis=1)
    g8 = jnp.concatenate([g, jnp.zeros((7, 2 * _H), _F32)], axis=0)
    h = _mlp3(g8, W0[...], b0[...], g0[...], p0[...], W1[...], b1[...],
              g1[...], p1[...], W2[...], b2[...])
    y = h[0:1, 0:1]
    out[...] = jnp.maximum(y, 0.0) + jnp.log1p(jnp.exp(-jnp.abs(y)))


def _head(nsum, esum, enc):
    ws = _enc_args(enc)
    full = [pl.BlockSpec(w.shape, lambda: (0, 0)) for w in ws]
    return pl.pallas_call(
        _head_kernel,
        in_specs=[pl.BlockSpec((8, _H), lambda: (0, 0)),
                  pl.BlockSpec((8, _H), lambda: (0, 0))] + full,
        out_specs=pl.BlockSpec((1, 1), lambda: (0, 0)),
        out_shape=jax.ShapeDtypeStruct((1, 1), _F32),
    )(nsum, esum, *ws)


def _sc_local_idx(tgt_v, idx_v, half0):
    def cidx(k, _):
        kk = pl.multiple_of(k * 16, 16)
        t16 = tgt_v[pl.ds(kk, 16)]
        loc = t16 - half0
        ok = (loc >= 0) & (loc < _HALF)
        idx_v[pl.ds(kk, 16)] = jnp.where(ok, loc, _TRASH0 + (t16 & 63))
        return 0
    lax.fori_loop(0, _CH // 16, cidx, 0)


def _sc_zero_acc(z_v, acc, s, width):
    zero = jnp.zeros((16,), _F32)

    def zfill(i, _):
        for j in range(width // 16):
            z_v[i, pl.ds(j * 16, 16)] = zero
        return 0
    lax.fori_loop(0, _ZR, zfill, 0)

    def zcp(t, _):
        pltpu.sync_copy(z_v, acc.at[pl.ds(s * _STRIPE + t * _ZR, _ZR)])
        return 0
    lax.fori_loop(0, _STRIPE // _ZR, zcp, 0)


def _sc_copy_out(acc, out_hbm, s, half0):
    row0 = s * _STRIPE

    @pl.when(s < 15)
    def _():
        pltpu.sync_copy(acc.at[pl.ds(row0, _STRIPE)],
                        out_hbm.at[pl.ds(half0 + row0, _STRIPE)])

    @pl.when(s == 15)
    def _():
        pltpu.sync_copy(acc.at[pl.ds(row0, _LAST)],
                        out_hbm.at[pl.ds(half0 + row0, _LAST)])


def _sc_layer_body(a_hbm, b_hbm, c_hbm, src_hbm, tgt_hbm, out_hbm,
                   src_v, tgt_v, idx_v, a_v, b_v, c_v, o_v, z_v, acc,
                   sem_i, sem_g, sem_c, sem_o):
    cc = lax.axis_index("c")
    s = lax.axis_index("s")
    half0 = cc * _HALF
    _sc_zero_acc(z_v, acc, s, _H)
    plsc.subcore_barrier()
    ebase = s * _EPW

    def in_sl(g):
        return pl.ds(ebase + g * _CH, _CH)

    def start_in(g, b):
        pltpu.async_copy(src_hbm.at[in_sl(g)], src_v[b], sem_i[b])
        pltpu.async_copy(tgt_hbm.at[in_sl(g)], tgt_v[b], sem_i[b])

    def wait_in(g, b):
        pltpu.make_async_copy(src_hbm.at[in_sl(g)], src_v[b],
                              sem_i[b]).wait()
        pltpu.make_async_copy(tgt_hbm.at[in_sl(g)], tgt_v[b],
                              sem_i[b]).wait()

    def start_fetch(g, b):
        pltpu.async_copy(a_hbm.at[src_v[b]], a_v[b], sem_g[b])
        pltpu.async_copy(b_hbm.at[tgt_v[b]], b_v[b], sem_g[b])
        pltpu.async_copy(c_hbm.at[in_sl(g)], c_v[b], sem_c[b])

    def wait_fetch(g, b):
        pltpu.make_async_copy(a_hbm.at[src_v[b]], a_v[b], sem_g[b]).wait()
        pltpu.make_async_copy(b_hbm.at[tgt_v[b]], b_v[b], sem_g[b]).wait()
        pltpu.make_async_copy(c_hbm.at[in_sl(g)], c_v[b], sem_c[b]).wait()

    def wait_scat(b):
        pltpu.make_async_copy(o_v[b], acc.at[idx_v[b]], sem_o[b]).wait()

    start_in(0, 0)
    start_in(1, 1)
    wait_in(0, 0)
    _sc_local_idx(tgt_v[0], idx_v[0], half0)
    start_fetch(0, 0)

    @pl.loop(0, _NCH, step=2)
    def _(g0):
        for b in range(2):
            g = g0 + b
            bn = 1 - b
            wait_fetch(g, b)

            @plsc.parallel_loop(0, _CH, step=1)
            def _(r):
                for j in range(2):
                    sl = pl.ds(j * 32, 32)
                    aw = plsc.bitcast(a_v[b][r, sl], jnp.int32)
                    bw = plsc.bitcast(b_v[b][r, sl], jnp.int32)
                    cw = plsc.bitcast(c_v[b][r, sl], jnp.int32)
                    lo = [plsc.bitcast(w << 16, _F32) for w in (aw, bw, cw)]
                    hi = [plsc.bitcast(w & jnp.int32(-65536), _F32)
                          for w in (aw, bw, cw)]
                    o_v[b][r, pl.ds(j * 32, 16)] = jnp.maximum(
                        lo[0] + lo[1] + lo[2], 0.0)
                    o_v[b][r, pl.ds(j * 32 + 16, 16)] = jnp.maximum(
                        hi[0] + hi[1] + hi[2], 0.0)

            pltpu.async_copy(o_v[b], acc.at[idx_v[b]], sem_o[b], add=True)

            @pl.when(g + 1 < _NCH)
            def _():
                wait_in(g + 1, bn)
                if b == 0:
                    @pl.when(g0 >= 2)
                    def _():
                        wait_scat(bn)
                else:
                    wait_scat(bn)
                _sc_local_idx(tgt_v[bn], idx_v[bn], half0)
                start_fetch(g + 1, bn)

            @pl.when(g + 2 < _NCH)
            def _():
                start_in(g + 2, b)

    wait_scat(0)
    wait_scat(1)
    plsc.subcore_barrier()
    _sc_copy_out(acc, out_hbm, s, half0)


def _sc_layer(A, B, C, src, tgt):
    mesh = plsc.VectorSubcoreMesh(core_axis_name="c", subcore_axis_name="s")
    fn = pl.kernel(
        _sc_layer_body,
        mesh=mesh,
        compiler_params=pltpu.CompilerParams(use_tc_tiling_on_sc=False,
                                             needs_layout_passes=False),
        out_type=jax.ShapeDtypeStruct((_N, _H), _F32),
        scratch_types=[
            [pltpu.VMEM((_CH,), jnp.int32)] * 2,
            [pltpu.VMEM((_CH,), jnp.int32)] * 2,
            [pltpu.VMEM((_CH,), jnp.int32)] * 2,
            [pltpu.VMEM((_CH, _H), _BF)] * 2,
            [pltpu.VMEM((_CH, _H), _BF)] * 2,
            [pltpu.VMEM((_CH, _H), _BF)] * 2,
            [pltpu.VMEM((_CH, _H), _F32)] * 2,
            pltpu.VMEM((_ZR, _H), _F32),
            pltpu.VMEM_SHARED((_HPAD, _H), _F32),
            [pltpu.SemaphoreType.DMA] * 2,
            [pltpu.SemaphoreType.DMA] * 2,
            [pltpu.SemaphoreType.DMA] * 2,
            [pltpu.SemaphoreType.DMA] * 2,
        ])
    return fn(A, B, C, src, tgt)


def _sc_deg_body(tgt_hbm, out_hbm, tgt_v, idx_v, val_v, z_v, acc,
                 sem_i, sem_o):
    cc = lax.axis_index("c")
    s = lax.axis_index("s")
    half0 = cc * _HALF
    _sc_zero_acc(z_v, acc, s, 16)
    one_hot = jnp.where(lax.iota(jnp.int32, 16) == 0,
                        jnp.float32(1.0), jnp.float32(0.0))

    def vfill(i, _):
        val_v[i, pl.ds(0, 16)] = one_hot
        return 0
    lax.fori_loop(0, _CH, vfill, 0)
    plsc.subcore_barrier()
    ebase = s * _EPW

    for b in range(2):
        pltpu.async_copy(tgt_hbm.at[pl.ds(ebase + b * _CH, _CH)], tgt_v[b],
                         sem_i[b])

    @pl.loop(0, _NCH, step=2)
    def _(g0):
        for b in range(2):
            g = g0 + b
            off = ebase + g * _CH
            pltpu.make_async_copy(tgt_hbm.at[pl.ds(off, _CH)], tgt_v[b],
                                  sem_i[b]).wait()

            @pl.when(g0 >= 2)
            def _():
                pltpu.make_async_copy(val_v, acc.at[idx_v[b]],
                                      sem_o[b]).wait()

            _sc_local_idx(tgt_v[b], idx_v[b], half0)
            pltpu.async_copy(val_v, acc.at[idx_v[b]], sem_o[b], add=True)

            @pl.when(g + 2 < _NCH)
            def _():
                pltpu.async_copy(tgt_hbm.at[pl.ds(off + 2 * _CH, _CH)],
                                 tgt_v[b], sem_i[b])

    for b in range(2):
        pltpu.make_async_copy(val_v, acc.at[idx_v[b]], sem_o[b]).wait()
    plsc.subcore_barrier()
    _sc_copy_out(acc, out_hbm, s, half0)


def _sc_deg(tgt):
    mesh = plsc.VectorSubcoreMesh(core_axis_name="c", subcore_axis_name="s")
    fn = pl.kernel(
        _sc_deg_body,
        mesh=mesh,
        compiler_params=pltpu.CompilerParams(use_tc_tiling_on_sc=False),
        out_type=jax.ShapeDtypeStruct((_N, 16), _F32),
        scratch_types=[
            [pltpu.VMEM((_CH,), jnp.int32)] * 2,
            [pltpu.VMEM((_CH,), jnp.int32)] * 2,
            pltpu.VMEM((_CH, 16), _F32),
            pltpu.VMEM((_ZR, 16), _F32),
            pltpu.VMEM_SHARED((_HPAD, 16), _F32),
            [pltpu.SemaphoreType.DMA] * 2,
            [pltpu.SemaphoreType.DMA] * 2,
        ])
    return fn(tgt)


def kernel(node_feats, edge_feats, params, edge_index):
    layers = params["layers"]
    npad = _EPAD - _E
    src = jnp.concatenate([edge_index[0], jnp.zeros((npad,), jnp.int32)])
    tgt_g = jnp.concatenate([edge_index[1], jnp.zeros((npad,), jnp.int32)])
    tgt_i = jnp.concatenate([edge_index[1],
                             jnp.full((npad,), -1, jnp.int32)])
    ef_pad = jnp.concatenate(
        [edge_feats, jnp.zeros((npad, 5), edge_feats.dtype)])
    Wms = [lp["Wm1"][0:_H] for lp in layers]
    Wmt = [lp["Wm1"][_H:2 * _H] for lp in layers]
    Wme = [lp["Wm1"][2 * _H:3 * _H] for lp in layers]
    bm1 = [lp["bm1"] for lp in layers]

    h, A, B = _node_enc(node_feats, params["node_enc"], Wms[0], Wmt[0])
    C0, C1, C2, esum = _edge_enc(ef_pad, params["edge_enc"], Wme, bm1)
    Cs = [C0, C1, C2]
    deg = _sc_deg(tgt_i)

    for l in range(3):
        S = _sc_layer(A, B, Cs[l], src, tgt_g)
        if l < 2:
            h, A, B = _update_mid(h, S, deg, layers[l], Wms[l + 1],
                                  Wmt[l + 1])
        else:
            nsum = _update_last(h, S, deg, layers[l])
    return _head(nsum, esum, params["head"])
